# Initial kernel scaffold; baseline (speedup 1.0000x reference)
import jax
import jax.numpy as jnp
from jax.experimental import pallas as pl


def kernel(wave, Win, b_in, lnA_g, lnA_b, Wqk, Wv, Wo, b_o, lnB_g, lnB_b, W1, b1, W2, b2, Wout, b_out):
    raise NotImplementedError("write your pallas kernel here")


# speedup vs baseline: 2.6716x; 2.6716x over previous
import functools

import jax
import jax.numpy as jnp
from jax import lax
from jax.experimental import pallas as pl
from jax.experimental.pallas import tpu as pltpu
from jax.experimental.pallas import tpu_sc as plsc

F32 = jnp.float32
BF16 = jnp.bfloat16
I32 = jnp.int32


def _lanesum(a):
    w = a.shape[-1]
    if w > 8:
        acc = a[:, 0:8]
        for i in range(1, w // 8):
            acc = acc + a[:, 8 * i:8 * (i + 1)]
        a = acc
        w = 8
    while w > 1:
        a = a[:, :w // 2] + a[:, w // 2:w]
        w //= 2
    return a


def _mxdot(a, b, dims=None):
    a = a.astype(BF16)
    b = b.astype(BF16)
    if dims is None:
        dims = (((a.ndim - 1,), (0,)), ((), ()))
    return lax.dot_general(a, b, dims, preferred_element_type=F32)

DM = 64
DEPTH = 4
HEADS = 4
DH = 16
BUCKET = 64
NHASH = 4
B = 16
L = 1280
BH = B * HEADS
NB = 20
NCH = 80
CBLK = 16
NCB = NCH // CBLK


def _embed_body(x_ref, w_ref, b_ref, o_ref):
    x = x_ref[0]
    o_ref[0] = jnp.dot(x, w_ref[...],
                       preferred_element_type=F32) + b_ref[...]


def _embed(xp, Win, b_in):
    return pl.pallas_call(
        _embed_body,
        grid=(B,),
        in_specs=[
            pl.BlockSpec((1, L, 2), lambda i: (i, 0, 0)),
            pl.BlockSpec((2, DM), lambda i: (0, 0)),
            pl.BlockSpec((1, DM), lambda i: (0, 0)),
        ],
        out_specs=pl.BlockSpec((1, L, DM), lambda i: (i, 0, 0)),
        out_shape=jax.ShapeDtypeStruct((B, L, DM), F32),
    )(xp, Win, b_in.reshape(1, DM))


def _proj_body(x_ref, g_ref, b_ref, wqk_ref, wv_ref, rot_ref,
               qk_ref, v_ref, bk_ref):
    x = x_ref[0]
    m = _lanesum(x) * (1.0 / DM)
    var = _lanesum((x - m) ** 2) * (1.0 / DM)
    ln = (x - m) / jnp.sqrt(var + 1e-5) * g_ref[...] + b_ref[...]
    qk = _mxdot(ln, wqk_ref[...])
    v = _mxdot(ln, wv_ref[...])
    rot = rot_ref[...]
    for h in range(HEADS):
        qk_h = qk[:, h * DH:(h + 1) * DH]
        v_h = v[:, h * DH:(h + 1) * DH]
        r = _mxdot(qk_h, rot)
        cols = []
        for i in range(NHASH):
            ri = r[:, i * NB:(i + 1) * NB]
            mx = jnp.max(ri, axis=-1, keepdims=True)
            il = lax.broadcasted_iota(I32, (L, NB), 1)
            cand = jnp.where(ri == mx, il, NB + 1)
            cols.append(jnp.min(cand, axis=-1, keepdims=True))
        qk_ref[h] = qk_h
        v_ref[h] = v_h
        bk_ref[h] = jnp.concatenate(cols, axis=-1)


def _proj(x2, g, b, Wqk, Wv, rotcat):
    return pl.pallas_call(
        _proj_body,
        grid=(B,),
        in_specs=[
            pl.BlockSpec((1, L, DM), lambda i: (i, 0, 0)),
            pl.BlockSpec((1, DM), lambda i: (0, 0)),
            pl.BlockSpec((1, DM), lambda i: (0, 0)),
            pl.BlockSpec((DM, DM), lambda i: (0, 0)),
            pl.BlockSpec((DM, DM), lambda i: (0, 0)),
            pl.BlockSpec((DH, NHASH * NB), lambda i: (0, 0)),
        ],
        out_specs=[
            pl.BlockSpec((HEADS, L, DH), lambda i: (i, 0, 0)),
            pl.BlockSpec((HEADS, L, DH), lambda i: (i, 0, 0)),
            pl.BlockSpec((HEADS, L, NHASH), lambda i: (i, 0, 0)),
        ],
        out_shape=[
            jax.ShapeDtypeStruct((BH, L, DH), F32),
            jax.ShapeDtypeStruct((BH, L, DH), F32),
            jax.ShapeDtypeStruct((BH, L, NHASH), I32),
        ],
    )(x2, g.reshape(1, DM), b.reshape(1, DM), Wqk, Wv, rotcat)


@functools.cache
def _sc_mesh():
    return plsc.VectorSubcoreMesh(core_axis_name="c", subcore_axis_name="s")


_GPB = 2


def _sort_body(bk_hbm, pidx_hbm, gdest_hbm, bk_v, pidx_v, gdest_v, cnt, run):
    wid = lax.axis_index("s") * 2 + lax.axis_index("c")
    bh0 = wid * _GPB
    pltpu.sync_copy(bk_hbm.at[pl.ds(bh0, _GPB)], bk_v)

    lane = lax.iota(I32, 16)
    active = lane < _GPB * NHASH
    g = jnp.where(active, lane // NHASH, 0)
    h = jnp.where(active, lane % NHASH, 0)
    tokbase = (bh0 + g) * L
    destbase = (bh0 + g) * (NHASH * L) + h * L

    for i in range(NB):
        cnt[i] = jnp.zeros((16,), I32)

    zero = jnp.zeros((16,), I32)

    def pass1(t, carry):
        ts = jnp.full((16,), t, I32)
        bkt = plsc.load_gather(bk_v, [g, ts, h], mask=active)
        bkt = jnp.where(active, bkt, 0)
        c = plsc.load_gather(cnt, [bkt, lane], mask=active)
        plsc.store_scatter(cnt, [bkt, lane], c + 1, mask=active)
        return carry

    lax.fori_loop(0, L, pass1, 0)

    acc = zero
    for i in range(NB):
        run[i] = acc
        acc = acc + cnt[i]

    def pass2(t, carry):
        ts = jnp.full((16,), t, I32)
        bkt = plsc.load_gather(bk_v, [g, ts, h], mask=active)
        bkt = jnp.where(active, bkt, 0)
        d = plsc.load_gather(run, [bkt, lane], mask=active)
        plsc.store_scatter(run, [bkt, lane], d + 1, mask=active)
        plsc.store_scatter(pidx_v, [g, h, d >> 7, d & 127], ts + tokbase,
                           mask=active)
        plsc.store_scatter(gdest_v, [g, h, ts >> 7, ts & 127], d + destbase,
                           mask=active)
        return carry

    lax.fori_loop(0, L, pass2, 0)

    pltpu.sync_copy(pidx_v, pidx_hbm.at[pl.ds(bh0, _GPB)])
    pltpu.sync_copy(gdest_v, gdest_hbm.at[pl.ds(bh0, _GPB)])


@functools.cache
def _sc_sort_call():
    return pl.kernel(
        _sort_body,
        out_type=[
            jax.ShapeDtypeStruct((BH, NHASH, L // 128, 128), I32),
            jax.ShapeDtypeStruct((BH, NHASH, L // 128, 128), I32),
        ],
        mesh=_sc_mesh(),
        scratch_types=[
            pltpu.VMEM((_GPB, L, NHASH), I32),
            pltpu.VMEM((_GPB, NHASH, L // 128, 128), I32),
            pltpu.VMEM((_GPB, NHASH, L // 128, 128), I32),
            pltpu.VMEM((NB, 16), I32),
            pltpu.VMEM((NB, 16), I32),
        ],
        compiler_params=pltpu.CompilerParams(
            use_tc_tiling_on_sc=False, needs_layout_passes=False),
    )


def _sc_sort(bk):
    return _sc_sort_call()(bk)


_NIDX = L // 128


def _gather_body(qk_hbm, v_hbm, pidx_hbm, sqk_hbm, sv_hbm, idx_v, rows_v, sem):
    wid = lax.axis_index("s") * 2 + lax.axis_index("c")
    items = BH * NHASH // 32

    def item(k, carry):
        it = wid * items + k
        bh = it // NHASH
        rnd = it % NHASH
        pltpu.sync_copy(pidx_hbm.at[bh, rnd], idx_v)
        for src, dst in ((qk_hbm, sqk_hbm), (v_hbm, sv_hbm)):
            cps = [pltpu.async_copy(src.at[idx_v.at[j]],
                                    rows_v.at[pl.ds(j * 128, 128)], sem)
                   for j in range(_NIDX)]
            for cp in cps:
                cp.wait()
            pltpu.sync_copy(rows_v, dst.at[bh, rnd])
        return carry

    lax.fori_loop(0, items, item, 0)


@functools.cache
def _sc_gather_call():
    return pl.kernel(
        _gather_body,
        out_type=[
            jax.ShapeDtypeStruct((BH, NHASH, L, DH), F32),
            jax.ShapeDtypeStruct((BH, NHASH, L, DH), F32),
        ],
        mesh=_sc_mesh(),
        scratch_types=[
            pltpu.VMEM((_NIDX, 128), I32),
            pltpu.VMEM((L, DH), F32),
            pltpu.SemaphoreType.DMA,
        ],
        compiler_params=pltpu.CompilerParams(
            use_tc_tiling_on_sc=False, needs_layout_passes=False),
    )


def _sc_gather(qk_flat, v_flat, pidx):
    return _sc_gather_call()(qk_flat, v_flat, pidx)


def _unsort_body(so_hbm, gdest_hbm, out_hbm, idx_v, rows_v, sem):
    wid = lax.axis_index("s") * 2 + lax.axis_index("c")
    items = BH * NHASH // 32

    def item(k, carry):
        it = wid * items + k
        bh = it // NHASH
        rnd = it % NHASH
        pltpu.sync_copy(gdest_hbm.at[bh, rnd], idx_v)
        cps = [pltpu.async_copy(so_hbm.at[idx_v.at[j]],
                                rows_v.at[pl.ds(j * 128, 128)], sem)
               for j in range(_NIDX)]
        for cp in cps:
            cp.wait()
        pltpu.sync_copy(rows_v, out_hbm.at[bh, rnd])
        return carry

    lax.fori_loop(0, items, item, 0)


@functools.cache
def _sc_unsort_call():
    return pl.kernel(
        _unsort_body,
        out_type=jax.ShapeDtypeStruct((BH, NHASH, L, 2 * DH), F32),
        mesh=_sc_mesh(),
        scratch_types=[
            pltpu.VMEM((_NIDX, 128), I32),
            pltpu.VMEM((L, 2 * DH), F32),
            pltpu.SemaphoreType.DMA,
        ],
        compiler_params=pltpu.CompilerParams(
            use_tc_tiling_on_sc=False, needs_layout_passes=False),
    )


def _sc_unsort(so_flat, gdest):
    return _sc_unsort_call()(so_flat, gdest)


def _attn_body(q_ref, kp_ref, v_ref, vp_ref, tq_ref, tk_ref, tp_ref, o_ref):
    scale = DH ** -0.5
    for j in range(CBLK):
        sl = pl.ds(j * BUCKET, BUCKET)
        q = q_ref[0, sl, :]
        if j == 0:
            kprev = kp_ref[0, 0]
            vprev = vp_ref[0, 0]
            tkp = tp_ref[0, 7, :]
        else:
            psl = pl.ds((j - 1) * BUCKET, BUCKET)
            kprev = q_ref[0, psl, :]
            vprev = v_ref[0, psl, :]
            tkp = tk_ref[0, j - 1, :]
        vcur = v_ref[0, sl, :]
        k_raw = jnp.concatenate([q, kprev], axis=0)
        kn = k_raw / (jnp.sqrt(_lanesum(k_raw * k_raw)) + 1e-8)
        v2 = jnp.concatenate([vcur, vprev], axis=0)
        dots = _mxdot(q, kn, (((1,), (1,)), ((), ()))) * scale
        tq = tq_ref[0, j]
        tk128 = jnp.concatenate([tk_ref[0, j, :], tkp],
                                axis=0).reshape(1, 2 * BUCKET)
        dots = jnp.where(tq == tk128, -5e4, dots)
        m = jnp.max(dots, axis=-1, keepdims=True)
        e = jnp.exp(dots - m)
        s = _lanesum(e)
        lse = jnp.log(s) + m
        probs = jnp.exp(dots - lse)
        o = _mxdot(probs, v2, (((1,), (0,)), ((), ())))
        o_ref[0, sl, :] = jnp.concatenate(
            [o, jnp.broadcast_to(lse, (BUCKET, DH))], axis=1)


def _attention(sqk, sv, pidx):
    sqk5 = sqk.reshape(BH, NHASH * L, DH)
    sv5 = sv.reshape(BH, NHASH * L, DH)
    sqk4 = sqk.reshape(BH, NCH, BUCKET, DH)
    sv4 = sv.reshape(BH, NCH, BUCKET, DH)
    tq4 = pidx.reshape(BH, NCH, BUCKET, 1)
    tk3 = pidx.reshape(BH, NCH, BUCKET)
    blk = CBLK * BUCKET
    return pl.pallas_call(
        _attn_body,
        grid=(BH, NCB),
        in_specs=[
            pl.BlockSpec((1, blk, DH), lambda i, c: (i, c, 0)),
            pl.BlockSpec((1, 1, BUCKET, DH),
                         lambda i, c: (i, (CBLK * c - 1) % NCH, 0, 0)),
            pl.BlockSpec((1, blk, DH), lambda i, c: (i, c, 0)),
            pl.BlockSpec((1, 1, BUCKET, DH),
                         lambda i, c: (i, (CBLK * c - 1) % NCH, 0, 0)),
            pl.BlockSpec((1, CBLK, BUCKET, 1), lambda i, c: (i, c, 0, 0)),
            pl.BlockSpec((1, CBLK, BUCKET), lambda i, c: (i, c, 0)),
            pl.BlockSpec((1, 8, BUCKET),
                         lambda i, c: (i, (2 * c - 1) % (NCH // 8), 0)),
        ],
        out_specs=pl.BlockSpec((1, blk, 2 * DH), lambda i, c: (i, c, 0)),
        out_shape=jax.ShapeDtypeStruct((BH, NHASH * L, 2 * DH), F32),
    )(sqk5, sqk4, sv5, sv4, tq4, tk3, tk3)


def _combine_body(o_ref, x1_ref, x2_ref, wo_ref, bo_ref, g_ref, b_ref,
                  w1_ref, b1_ref, w2_ref, b2_ref, y1_ref, y2_ref):
    outs = []
    for h in range(HEADS):
        ob = o_ref[h]
        o_h = ob[:, :, 0:DH]
        l_h = ob[:, :, DH:DH + 1]
        m = jnp.max(l_h, axis=0, keepdims=True)
        e = jnp.exp(l_h - m)
        sm = jnp.sum(e, axis=0, keepdims=True)
        lse4 = jnp.log(sm) + m
        w = jnp.exp(l_h - lse4)
        outs.append(jnp.sum(o_h * w, axis=0))
    cat = jnp.concatenate(outs, axis=-1)
    attn = _mxdot(cat, wo_ref[...]) + bo_ref[...]
    y1 = x1_ref[0] + attn
    m2 = _lanesum(y1) * (1.0 / DM)
    var2 = _lanesum((y1 - m2) ** 2) * (1.0 / DM)
    hf = (y1 - m2) / jnp.sqrt(var2 + 1e-5) * g_ref[...] + b_ref[...]
    f1 = jax.nn.gelu(_mxdot(hf, w1_ref[...]) + b1_ref[...])
    ffn = _mxdot(f1, w2_ref[...]) + b2_ref[...]
    y1_ref[0] = y1
    y2_ref[0] = x2_ref[0] + ffn


def _combine(o_un, x1, x2, Wo, bo, g, b, W1, b1, W2, b2):
    return pl.pallas_call(
        _combine_body,
        grid=(B,),
        in_specs=[
            pl.BlockSpec((HEADS, NHASH, L, 2 * DH), lambda i: (i, 0, 0, 0)),
            pl.BlockSpec((1, L, DM), lambda i: (i, 0, 0)),
            pl.BlockSpec((1, L, DM), lambda i: (i, 0, 0)),
            pl.BlockSpec((DM, DM), lambda i: (0, 0)),
            pl.BlockSpec((1, DM), lambda i: (0, 0)),
            pl.BlockSpec((1, DM), lambda i: (0, 0)),
            pl.BlockSpec((1, DM), lambda i: (0, 0)),
            pl.BlockSpec((DM, 4 * DM), lambda i: (0, 0)),
            pl.BlockSpec((1, 4 * DM), lambda i: (0, 0)),
            pl.BlockSpec((4 * DM, DM), lambda i: (0, 0)),
            pl.BlockSpec((1, DM), lambda i: (0, 0)),
        ],
        out_specs=[
            pl.BlockSpec((1, L, DM), lambda i: (i, 0, 0)),
            pl.BlockSpec((1, L, DM), lambda i: (i, 0, 0)),
        ],
        out_shape=[
            jax.ShapeDtypeStruct((B, L, DM), F32),
            jax.ShapeDtypeStruct((B, L, DM), F32),
        ],
    )(o_un, x1, x2, Wo, bo.reshape(1, DM), g.reshape(1, DM),
      b.reshape(1, DM), W1, b1.reshape(1, 4 * DM), W2, b2.reshape(1, DM))


def _head_body(x1_ref, x2_ref, w_ref, b_ref, o_ref):
    hfin = (x1_ref[0] + x2_ref[0]) * 0.5
    o_ref[0] = jnp.dot(hfin, w_ref[...],
                       preferred_element_type=F32) + b_ref[...]


def _head(x1, x2, Wout, b_out):
    return pl.pallas_call(
        _head_body,
        grid=(B,),
        in_specs=[
            pl.BlockSpec((1, L, DM), lambda i: (i, 0, 0)),
            pl.BlockSpec((1, L, DM), lambda i: (i, 0, 0)),
            pl.BlockSpec((DM, 1), lambda i: (0, 0)),
            pl.BlockSpec((1, 1), lambda i: (0, 0)),
        ],
        out_specs=pl.BlockSpec((1, L, 1), lambda i: (i, 0, 0)),
        out_shape=jax.ShapeDtypeStruct((B, L, 1), F32),
    )(x1, x2, Wout, b_out.reshape(1, 1))


def kernel(wave, Win, b_in, lnA_g, lnA_b, Wqk, Wv, Wo, b_o, lnB_g, lnB_b,
           W1, b1, W2, b2, Wout, b_out):
    x = jnp.transpose(wave, (0, 2, 1))
    pad = L - x.shape[1]
    xp = jnp.concatenate([x, jnp.zeros((B, pad, 2), F32)], axis=1)
    h = _embed(xp, Win, b_in)
    x1 = x2 = h
    for i in range(DEPTH):
        rot = jax.random.normal(
            jax.random.fold_in(jax.random.key(42), i),
            (DH, NHASH, NB // 2), dtype=F32)
        rotcat = jnp.concatenate([rot, -rot], axis=-1).reshape(DH, NHASH * NB)
        qk, v, bk = _proj(x2, lnA_g[i], lnA_b[i], Wqk[i], Wv[i], rotcat)
        pidx, gdest = _sc_sort(bk)
        sqk, sv = _sc_gather(qk.reshape(BH * L, DH), v.reshape(BH * L, DH),
                             pidx)
        so = _attention(sqk, sv, pidx)
        o_un = _sc_unsort(so.reshape(BH * NHASH * L, 2 * DH), gdest)
        y1, y2 = _combine(o_un, x1, x2, Wo[i], b_o[i], lnB_g[i], lnB_b[i],
                          W1[i], b1[i], W2[i], b2[i])
        x1, x2 = y1, y2
    out = _head(x1, x2, Wout, b_out)
    return out[:, :1250, 0]
